# baseline (speedup 1.0000x reference)
import functools
import math

import jax
import jax.numpy as jnp
from jax import lax
from jax.experimental import pallas as pl
from jax.experimental.pallas import tpu as pltpu
from jax.experimental.pallas import tpu_sc as plsc

_B = 2
_N = 2048
_T = _B * _N
_D = 1024
_E = 16
_K = 2
_H = 2048
_BAL = 0.01

_M = 512
_RMAX = _T * _K + _E * _M
_NT = _RMAX // _M
_TT = 512
_NTT = _T // _TT


def _gelu(v):
    return 0.5 * v * (1.0 + lax.erf(v * (1.0 / math.sqrt(2.0))))


def _cumsum(a, axis):
    n = a.shape[axis]
    s = 1
    while s < n:
        zshape = list(a.shape)
        zshape[axis] = s
        sl = [slice(None)] * a.ndim
        sl[axis] = slice(0, n - s)
        a = a + jnp.concatenate(
            [jnp.zeros(zshape, a.dtype), a[tuple(sl)]], axis=axis)
        s *= 2
    return a


def _router_body(x_ref, gw_ref, i1_ref, i2_ref, s1_ref, s2_ref, pp_ref):
    x = x_ref[...]
    gw = gw_ref[...]
    logits = lax.dot_general(x, gw, (((1,), (1,)), ((), ())),
                             preferred_element_type=jnp.float32)
    m = jnp.max(logits, axis=1, keepdims=True)
    ex = jnp.exp(logits - m)
    probs = ex / jnp.sum(ex, axis=1, keepdims=True)
    lane = lax.broadcasted_iota(jnp.int32, (_TT, _E), 1)
    m1 = jnp.max(probs, axis=1, keepdims=True)
    i1 = jnp.min(jnp.where(probs == m1, lane, _E), axis=1, keepdims=True)
    p2 = jnp.where(lane == i1, -1.0, probs)
    m2 = jnp.max(p2, axis=1, keepdims=True)
    i2 = jnp.min(jnp.where(p2 == m2, lane, _E), axis=1, keepdims=True)
    i1_ref[...] = jnp.broadcast_to(i1, (_TT, 8))
    i2_ref[...] = jnp.broadcast_to(i2, (_TT, 8))
    s1_ref[...] = jnp.broadcast_to(m1, (_TT, 16))
    s2_ref[...] = jnp.broadcast_to(m2, (_TT, 16))
    pp_ref[...] = jnp.sum(probs, axis=0).reshape(1, 1, _E)


def _router(x_flat, gate_w):
    return pl.pallas_call(
        _router_body,
        grid=(_NTT,),
        in_specs=[
            pl.BlockSpec((_TT, _D), lambda i: (i, 0)),
            pl.BlockSpec((_E, _D), lambda i: (0, 0)),
        ],
        out_specs=[
            pl.BlockSpec((_TT, 8), lambda i: (i, 0)),
            pl.BlockSpec((_TT, 8), lambda i: (i, 0)),
            pl.BlockSpec((_TT, 16), lambda i: (i, 0)),
            pl.BlockSpec((_TT, 16), lambda i: (i, 0)),
            pl.BlockSpec((1, 1, _E), lambda i: (i, 0, 0)),
        ],
        out_shape=[
            jax.ShapeDtypeStruct((_T, 8), jnp.int32),
            jax.ShapeDtypeStruct((_T, 8), jnp.int32),
            jax.ShapeDtypeStruct((_T, 16), jnp.float32),
            jax.ShapeDtypeStruct((_T, 16), jnp.float32),
            jax.ShapeDtypeStruct((_NTT, 1, _E), jnp.float32),
        ],
    )(x_flat, gate_w)


def _plan_body(i1_ref, i2_ref, pp_ref,
               dst1_ref, dst2_ref, te_ref, nv_ref, freq_ref, prob_ref,
               loss_ref):
    e1 = i1_ref[:, 0:1]
    e2 = i2_ref[:, 0:1]
    lane = lax.broadcasted_iota(jnp.int32, (_T, _E), 1)
    oh1 = (e1 == lane).astype(jnp.int32)
    oh2 = (e2 == lane).astype(jnp.int32)
    c1 = _cumsum(oh1, axis=0)
    counts1 = c1[_T - 1:_T, :]
    c2 = _cumsum(oh2, axis=0)
    counts = counts1 + c2[_T - 1:_T, :]
    rank1 = c1 - oh1
    rank2 = counts1 + c2 - oh2
    pad = ((counts + (_M - 1)) // _M) * _M
    bnd = _cumsum(pad, axis=1)
    off = bnd - pad
    total = bnd[0:1, _E - 1:_E]
    dst1_ref[...] = jnp.sum(oh1 * (off + rank1), axis=1, keepdims=True)
    dst2_ref[...] = jnp.sum(oh2 * (off + rank2), axis=1, keepdims=True)
    tl = lax.broadcasted_iota(jnp.int32, (_NT, _E), 0) * _M
    te = jnp.sum((tl >= bnd).astype(jnp.int32), axis=1, keepdims=True)
    te_ref[...] = jnp.minimum(te, _E - 1)
    nv_ref[...] = total
    freq = counts.astype(jnp.float32) * (1.0 / (_T * _K))
    freq_ref[...] = freq
    prob = jnp.sum(pp_ref[...], axis=0) * (1.0 / _T)
    prob_ref[...] = prob
    loss_ref[...] = _BAL * jnp.sum(freq * prob, axis=1, keepdims=True)


def _plan(i1c, i2c, pp):
    return pl.pallas_call(
        _plan_body,
        grid=(1,),
        in_specs=[
            pl.BlockSpec((_T, 8), lambda i: (0, 0)),
            pl.BlockSpec((_T, 8), lambda i: (0, 0)),
            pl.BlockSpec((_NTT, 1, _E), lambda i: (0, 0, 0)),
        ],
        out_specs=[
            pl.BlockSpec((_T, 1), lambda i: (0, 0)),
            pl.BlockSpec((_T, 1), lambda i: (0, 0)),
            pl.BlockSpec((_NT, 1), lambda i: (0, 0)),
            pl.BlockSpec((1, 1), lambda i: (0, 0)),
            pl.BlockSpec((1, _E), lambda i: (0, 0)),
            pl.BlockSpec((1, _E), lambda i: (0, 0)),
            pl.BlockSpec((1, 1), lambda i: (0, 0)),
        ],
        out_shape=[
            jax.ShapeDtypeStruct((_T, 1), jnp.int32),
            jax.ShapeDtypeStruct((_T, 1), jnp.int32),
            jax.ShapeDtypeStruct((_NT, 1), jnp.int32),
            jax.ShapeDtypeStruct((1, 1), jnp.int32),
            jax.ShapeDtypeStruct((1, _E), jnp.float32),
            jax.ShapeDtypeStruct((1, _E), jnp.float32),
            jax.ShapeDtypeStruct((1, 1), jnp.float32),
        ],
    )(i1c, i2c, pp)


def _expert_body(te_ref, nv_ref, gx_ref, w1_ref, b1_ref, w2_ref, b2_ref,
                 gy_ref):
    i = pl.program_id(0)

    @pl.when(i * _M < nv_ref[0])
    def _():
        x = gx_ref[...]
        w1 = w1_ref[0]
        h = lax.dot_general(x, w1, (((1,), (1,)), ((), ())),
                            preferred_element_type=jnp.float32)
        h = _gelu(h + b1_ref[0])
        w2 = w2_ref[0]
        y = lax.dot_general(h, w2, (((1,), (1,)), ((), ())),
                            preferred_element_type=jnp.float32)
        gy_ref[...] = y + b2_ref[0]


def _expert_mlp(tile_expert, nvalid, gx, w1, b1r, w2, b2r):
    def _row_idx(i, te, nv):
        return (jnp.minimum(i, nv[0] // _M - 1), 0)

    grid_spec = pltpu.PrefetchScalarGridSpec(
        num_scalar_prefetch=2,
        grid=(_NT,),
        in_specs=[
            pl.BlockSpec((_M, _D), _row_idx),
            pl.BlockSpec((1, _H, _D), lambda i, te, nv: (te[i], 0, 0)),
            pl.BlockSpec((1, 1, _H), lambda i, te, nv: (te[i], 0, 0)),
            pl.BlockSpec((1, _D, _H), lambda i, te, nv: (te[i], 0, 0)),
            pl.BlockSpec((1, 1, _D), lambda i, te, nv: (te[i], 0, 0)),
        ],
        out_specs=pl.BlockSpec((_M, _D), _row_idx),
    )
    return pl.pallas_call(
        _expert_body,
        grid_spec=grid_spec,
        out_shape=jax.ShapeDtypeStruct((_RMAX, _D), jnp.float32),
    )(tile_expert, nvalid, gx, w1, b1r, w2, b2r)


_NW = 32
_TPW = _T // _NW
_CH = 32


def _sc_mesh():
    return plsc.VectorSubcoreMesh(core_axis_name="c", subcore_axis_name="s")


def _sc_dispatch(x_flat, dst1, dst2):
    @functools.partial(
        pl.kernel, mesh=_sc_mesh(),
        out_type=jax.ShapeDtypeStruct((_RMAX, _D), jnp.float32),
        scratch_types=[
            pltpu.VMEM((_CH,), jnp.int32),
            pltpu.VMEM((_CH,), jnp.int32),
            pltpu.VMEM((_CH, _D), jnp.float32),
            pltpu.SemaphoreType.DMA,
            pltpu.SemaphoreType.DMA,
        ],
    )
    def k(x_hbm, d1_hbm, d2_hbm, gx_hbm, i1_v, i2_v, rows_v, sem1, sem2):
        wid = lax.axis_index("s") * 2 + lax.axis_index("c")
        base = wid * _TPW

        def body(j, carry):
            off = base + j * _CH
            pltpu.sync_copy(x_hbm.at[pl.ds(off, _CH)], rows_v)
            pltpu.sync_copy(d1_hbm.at[pl.ds(off, _CH)], i1_v)
            pltpu.sync_copy(d2_hbm.at[pl.ds(off, _CH)], i2_v)
            c1 = pltpu.make_async_copy(rows_v, gx_hbm.at[i1_v], sem1)
            c2 = pltpu.make_async_copy(rows_v, gx_hbm.at[i2_v], sem2)
            c1.start()
            c2.start()
            c1.wait()
            c2.wait()
            return carry

        lax.fori_loop(0, _TPW // _CH, body, 0)

    return k(x_flat, dst1, dst2)


_CCH = 8
_NCH = _TPW // _CCH
_ROWB = _CCH * _D


def _sc_combine(gy, dst1, dst2, sh_flat, s1_flat, s2_flat):
    @functools.partial(
        pl.kernel, mesh=_sc_mesh(),
        out_type=jax.ShapeDtypeStruct((_T * _D,), jnp.float32),
        scratch_types=[
            pltpu.VMEM((_TPW,), jnp.int32),
            pltpu.VMEM((_TPW,), jnp.int32),
            pltpu.VMEM((_TPW * 16,), jnp.float32),
            pltpu.VMEM((_TPW * 16,), jnp.float32),
            pltpu.VMEM((_CCH, _D), jnp.float32),
            pltpu.VMEM((_CCH, _D), jnp.float32),
            pltpu.VMEM((_CCH, _D), jnp.float32),
            pltpu.VMEM((_CCH, _D), jnp.float32),
            pltpu.VMEM((_ROWB,), jnp.float32),
            pltpu.VMEM((_ROWB,), jnp.float32),
            pltpu.VMEM((_ROWB,), jnp.float32),
            pltpu.VMEM((_ROWB,), jnp.float32),
            pltpu.SemaphoreType.DMA,
            pltpu.SemaphoreType.DMA,
            pltpu.SemaphoreType.DMA,
            pltpu.SemaphoreType.DMA,
        ],
    )
    def k(gy_hbm, d1_hbm, d2_hbm, sh_hbm, s1_hbm, s2_hbm, out_hbm,
          i1_v, i2_v, s1_v, s2_v, r1a, r1b, r2a, r2b, sha, shb, oa, ob,
          semla, semlb, semoa, semob):
        wid = lax.axis_index("s") * 2 + lax.axis_index("c")
        base = wid * _TPW
        pltpu.sync_copy(d1_hbm.at[pl.ds(base, _TPW)], i1_v)
        pltpu.sync_copy(d2_hbm.at[pl.ds(base, _TPW)], i2_v)
        pltpu.sync_copy(s1_hbm.at[pl.ds(base * 16, _TPW * 16)], s1_v)
        pltpu.sync_copy(s2_hbm.at[pl.ds(base * 16, _TPW * 16)], s2_v)
        bufs = ((r1a, r2a, sha, oa, semla, semoa),
                (r1b, r2b, shb, ob, semlb, semob))

        def load_descs(j, r1x, r2x, shx, seml):
            return (
                pltpu.make_async_copy(
                    gy_hbm.at[i1_v.at[pl.ds(j * _CCH, _CCH)]], r1x, seml),
                pltpu.make_async_copy(
                    gy_hbm.at[i2_v.at[pl.ds(j * _CCH, _CCH)]], r2x, seml),
                pltpu.make_async_copy(
                    sh_hbm.at[pl.ds((base + j * _CCH) * _D, _ROWB)],
                    shx, seml),
            )

        def start_loads(j, r1x, r2x, shx, seml):
            for c in load_descs(j, r1x, r2x, shx, seml):
                c.start()

        def wait_loads(r1x, r2x, shx, seml):
            for c in load_descs(0, r1x, r2x, shx, seml):
                c.wait()

        def out_desc(j, ox, semo):
            return pltpu.make_async_copy(
                ox, out_hbm.at[pl.ds((base + j * _CCH) * _D, _ROWB)], semo)

        for b in range(2):
            start_loads(b, *bufs[b][:3], bufs[b][4])

        @pl.loop(0, _NCH, step=2)
        def _chunks(j0):
            for b in range(2):
                j = j0 + b
                r1x, r2x, shx, ox, seml, semo = bufs[b]
                wait_loads(r1x, r2x, shx, seml)

                @pl.when(j >= 2)
                def _():
                    out_desc(0, ox, semo).wait()

                def row(i, carry):
                    sb = (j * _CCH + i) * 16
                    s1vec = s1_v[pl.ds(sb, 16)]
                    s2vec = s2_v[pl.ds(sb, 16)]
                    for g in range(_D // 16):
                        sl = pl.ds(g * 16, 16)
                        fl = pl.ds(i * _D + g * 16, 16)
                        ox[fl] = (shx[fl] + s1vec * r1x[i, sl]
                                  + s2vec * r2x[i, sl])
                    return carry

                lax.fori_loop(0, _CCH, row, 0)
                out_desc(j, ox, semo).start()

                @pl.when(j + 2 < _NCH)
                def _():
                    start_loads(j + 2, r1x, r2x, shx, seml)

        for b in range(2):
            out_desc(0, bufs[b][3], bufs[b][5]).wait()

    return k(gy, dst1, dst2, sh_flat, s1_flat, s2_flat)


def _shared_body(x_ref, w1_ref, b1_ref, w2_ref, b2_ref, o_ref):
    x = x_ref[...]
    h = lax.dot_general(x, w1_ref[...], (((1,), (1,)), ((), ())),
                        preferred_element_type=jnp.float32)
    h = _gelu(h + b1_ref[...])
    sh = lax.dot_general(h, w2_ref[...], (((1,), (1,)), ((), ())),
                         preferred_element_type=jnp.float32)
    o_ref[...] = sh + b2_ref[...]


def _shared(x_flat, sw1, sb1, sw2, sb2):
    return pl.pallas_call(
        _shared_body,
        grid=(_NTT,),
        in_specs=[
            pl.BlockSpec((_TT, _D), lambda i: (i, 0)),
            pl.BlockSpec((_H, _D), lambda i: (0, 0)),
            pl.BlockSpec((1, _H), lambda i: (0, 0)),
            pl.BlockSpec((_D, _H), lambda i: (0, 0)),
            pl.BlockSpec((1, _D), lambda i: (0, 0)),
        ],
        out_specs=pl.BlockSpec((_TT, _D), lambda i: (i, 0)),
        out_shape=jax.ShapeDtypeStruct((_T, _D), jnp.float32),
    )(x_flat, sw1, sb1, sw2, sb2)


def kernel(x, shared_fc1_w, shared_fc1_b, shared_fc2_w, shared_fc2_b,
           expert_fc1_w, expert_fc1_b, expert_fc2_w, expert_fc2_b, gate_w):
    x_flat = x.reshape(_T, _D)
    i1c, i2c, s1x, s2x, pp = _router(x_flat, gate_w)
    dst1, dst2, te, nv, freq, prob, loss = _plan(i1c, i2c, pp)
    dst1 = dst1.reshape(_T)
    dst2 = dst2.reshape(_T)

    gx = _sc_dispatch(x_flat, dst1, dst2)

    sh = _shared(x_flat, shared_fc1_w, shared_fc1_b.reshape(1, _H),
                 shared_fc2_w, shared_fc2_b.reshape(1, _D))

    gy = _expert_mlp(
        te.reshape(_NT), nv.reshape(1), gx,
        expert_fc1_w, expert_fc1_b.reshape(_E, 1, _H),
        expert_fc2_w, expert_fc2_b.reshape(_E, 1, _D))

    out = _sc_combine(gy, dst1, dst2, sh.reshape(_T * _D),
                      s1x.reshape(_T * 16), s2x.reshape(_T * 16))
    return (out.reshape(_B, _N, _D), loss.reshape(()), freq.reshape(_E),
            prob.reshape(_E))


# speedup vs baseline: 1.1737x; 1.1737x over previous
import functools
import math

import jax
import jax.numpy as jnp
from jax import lax
from jax.experimental import pallas as pl
from jax.experimental.pallas import tpu as pltpu
from jax.experimental.pallas import tpu_sc as plsc

_B = 2
_N = 2048
_T = _B * _N
_D = 1024
_E = 16
_K = 2
_H = 2048
_BAL = 0.01

_M = 512
_RMAX = _T * _K + _E * _M
_NT = _RMAX // _M
_TT = 512
_NTT = _T // _TT


def _gelu(v):
    return 0.5 * v * (1.0 + lax.erf(v * (1.0 / math.sqrt(2.0))))


def _cumsum(a, axis):
    n = a.shape[axis]
    s = 1
    while s < n:
        zshape = list(a.shape)
        zshape[axis] = s
        sl = [slice(None)] * a.ndim
        sl[axis] = slice(0, n - s)
        a = a + jnp.concatenate(
            [jnp.zeros(zshape, a.dtype), a[tuple(sl)]], axis=axis)
        s *= 2
    return a


def _router_body(x_ref, gw_ref, i1_ref, i2_ref, s1_ref, s2_ref, pp_ref):
    x = x_ref[...]
    gw = gw_ref[...]
    logits = lax.dot_general(x, gw, (((1,), (1,)), ((), ())),
                             preferred_element_type=jnp.float32)
    m = jnp.max(logits, axis=1, keepdims=True)
    ex = jnp.exp(logits - m)
    probs = ex / jnp.sum(ex, axis=1, keepdims=True)
    lane = lax.broadcasted_iota(jnp.int32, (_TT, _E), 1)
    m1 = jnp.max(probs, axis=1, keepdims=True)
    i1 = jnp.min(jnp.where(probs == m1, lane, _E), axis=1, keepdims=True)
    p2 = jnp.where(lane == i1, -1.0, probs)
    m2 = jnp.max(p2, axis=1, keepdims=True)
    i2 = jnp.min(jnp.where(p2 == m2, lane, _E), axis=1, keepdims=True)
    i1_ref[...] = jnp.broadcast_to(i1, (_TT, 8))
    i2_ref[...] = jnp.broadcast_to(i2, (_TT, 8))
    s1_ref[...] = jnp.broadcast_to(m1, (_TT, 16))
    s2_ref[...] = jnp.broadcast_to(m2, (_TT, 16))
    pp_ref[...] = jnp.sum(probs, axis=0).reshape(1, 1, _E)


def _router(x_flat, gate_w):
    return pl.pallas_call(
        _router_body,
        grid=(_NTT,),
        in_specs=[
            pl.BlockSpec((_TT, _D), lambda i: (i, 0)),
            pl.BlockSpec((_E, _D), lambda i: (0, 0)),
        ],
        out_specs=[
            pl.BlockSpec((_TT, 8), lambda i: (i, 0)),
            pl.BlockSpec((_TT, 8), lambda i: (i, 0)),
            pl.BlockSpec((_TT, 16), lambda i: (i, 0)),
            pl.BlockSpec((_TT, 16), lambda i: (i, 0)),
            pl.BlockSpec((1, 1, _E), lambda i: (i, 0, 0)),
        ],
        out_shape=[
            jax.ShapeDtypeStruct((_T, 8), jnp.int32),
            jax.ShapeDtypeStruct((_T, 8), jnp.int32),
            jax.ShapeDtypeStruct((_T, 16), jnp.float32),
            jax.ShapeDtypeStruct((_T, 16), jnp.float32),
            jax.ShapeDtypeStruct((_NTT, 1, _E), jnp.float32),
        ],
    )(x_flat, gate_w)


def _plan_body(i1_ref, i2_ref, pp_ref,
               dst1_ref, dst2_ref, te_ref, nv_ref, freq_ref, prob_ref,
               loss_ref):
    e1 = i1_ref[:, 0:1]
    e2 = i2_ref[:, 0:1]
    lane = lax.broadcasted_iota(jnp.int32, (_T, _E), 1)
    oh1 = (e1 == lane).astype(jnp.int32)
    oh2 = (e2 == lane).astype(jnp.int32)
    c1 = _cumsum(oh1, axis=0)
    counts1 = c1[_T - 1:_T, :]
    c2 = _cumsum(oh2, axis=0)
    counts = counts1 + c2[_T - 1:_T, :]
    rank1 = c1 - oh1
    rank2 = counts1 + c2 - oh2
    pad = ((counts + (_M - 1)) // _M) * _M
    bnd = _cumsum(pad, axis=1)
    off = bnd - pad
    total = bnd[0:1, _E - 1:_E]
    dst1_ref[...] = jnp.sum(oh1 * (off + rank1), axis=1, keepdims=True)
    dst2_ref[...] = jnp.sum(oh2 * (off + rank2), axis=1, keepdims=True)
    tl = lax.broadcasted_iota(jnp.int32, (_NT, _E), 0) * _M
    te = jnp.sum((tl >= bnd).astype(jnp.int32), axis=1, keepdims=True)
    te_ref[...] = jnp.minimum(te, _E - 1)
    nv_ref[...] = total
    freq = counts.astype(jnp.float32) * (1.0 / (_T * _K))
    freq_ref[...] = freq
    prob = jnp.sum(pp_ref[...], axis=0) * (1.0 / _T)
    prob_ref[...] = prob
    loss_ref[...] = _BAL * jnp.sum(freq * prob, axis=1, keepdims=True)


def _plan(i1c, i2c, pp):
    return pl.pallas_call(
        _plan_body,
        grid=(1,),
        in_specs=[
            pl.BlockSpec((_T, 8), lambda i: (0, 0)),
            pl.BlockSpec((_T, 8), lambda i: (0, 0)),
            pl.BlockSpec((_NTT, 1, _E), lambda i: (0, 0, 0)),
        ],
        out_specs=[
            pl.BlockSpec((_T, 1), lambda i: (0, 0)),
            pl.BlockSpec((_T, 1), lambda i: (0, 0)),
            pl.BlockSpec((_NT, 1), lambda i: (0, 0)),
            pl.BlockSpec((1, 1), lambda i: (0, 0)),
            pl.BlockSpec((1, _E), lambda i: (0, 0)),
            pl.BlockSpec((1, _E), lambda i: (0, 0)),
            pl.BlockSpec((1, 1), lambda i: (0, 0)),
        ],
        out_shape=[
            jax.ShapeDtypeStruct((_T, 1), jnp.int32),
            jax.ShapeDtypeStruct((_T, 1), jnp.int32),
            jax.ShapeDtypeStruct((_NT, 1), jnp.int32),
            jax.ShapeDtypeStruct((1, 1), jnp.int32),
            jax.ShapeDtypeStruct((1, _E), jnp.float32),
            jax.ShapeDtypeStruct((1, _E), jnp.float32),
            jax.ShapeDtypeStruct((1, 1), jnp.float32),
        ],
    )(i1c, i2c, pp)


def _expert_body(te_ref, nv_ref, gx_ref, w1_ref, b1_ref, w2_ref, b2_ref,
                 gy_ref):
    i = pl.program_id(0)

    @pl.when(i * _M < nv_ref[0])
    def _():
        x = gx_ref[...]
        w1 = w1_ref[0]
        h = lax.dot_general(x, w1, (((1,), (1,)), ((), ())),
                            preferred_element_type=jnp.float32)
        h = _gelu(h + b1_ref[0])
        w2 = w2_ref[0]
        y = lax.dot_general(h, w2, (((1,), (1,)), ((), ())),
                            preferred_element_type=jnp.float32)
        gy_ref[...] = y + b2_ref[0]


def _expert_mlp(tile_expert, nvalid, gx, w1, b1r, w2, b2r):
    def _row_idx(i, te, nv):
        return (jnp.minimum(i, nv[0] // _M - 1), 0)

    grid_spec = pltpu.PrefetchScalarGridSpec(
        num_scalar_prefetch=2,
        grid=(_NT,),
        in_specs=[
            pl.BlockSpec((_M, _D), _row_idx),
            pl.BlockSpec((1, _H, _D), lambda i, te, nv: (te[i], 0, 0)),
            pl.BlockSpec((1, 1, _H), lambda i, te, nv: (te[i], 0, 0)),
            pl.BlockSpec((1, _D, _H), lambda i, te, nv: (te[i], 0, 0)),
            pl.BlockSpec((1, 1, _D), lambda i, te, nv: (te[i], 0, 0)),
        ],
        out_specs=pl.BlockSpec((_M, _D), _row_idx),
    )
    return pl.pallas_call(
        _expert_body,
        grid_spec=grid_spec,
        out_shape=jax.ShapeDtypeStruct((_RMAX, _D), jnp.float32),
    )(tile_expert, nvalid, gx, w1, b1r, w2, b2r)


_NW = 32
_TPW = _T // _NW
_CH = 32


def _sc_mesh():
    return plsc.VectorSubcoreMesh(core_axis_name="c", subcore_axis_name="s")


_NCHD = _TPW // _CH


def _sc_dispatch(x_flat, d1m, d2m):
    @functools.partial(
        pl.kernel, mesh=_sc_mesh(),
        out_type=jax.ShapeDtypeStruct((_RMAX, _D), jnp.float32),
        scratch_types=[
            pltpu.VMEM((_NCHD, _CH), jnp.int32),
            pltpu.VMEM((_NCHD, _CH), jnp.int32),
            pltpu.VMEM((_CH, _D), jnp.float32),
            pltpu.VMEM((_CH, _D), jnp.float32),
            pltpu.SemaphoreType.DMA,
            pltpu.SemaphoreType.DMA,
            pltpu.SemaphoreType.DMA,
            pltpu.SemaphoreType.DMA,
        ],
    )
    def k(x_hbm, d1_hbm, d2_hbm, gx_hbm, i1m, i2m, ra, rb,
          semla, semlb, semsa, semsb):
        wid = lax.axis_index("s") * 2 + lax.axis_index("c")
        base = wid * _TPW
        pltpu.sync_copy(d1_hbm.at[wid], i1m)
        pltpu.sync_copy(d2_hbm.at[wid], i2m)
        rbufs = (ra, rb)
        seml = (semla, semlb)
        sems = (semsa, semsb)

        def load_desc(j, b):
            return pltpu.make_async_copy(
                x_hbm.at[pl.ds(base + j * _CH, _CH)], rbufs[b], seml[b])

        def scat_descs(j, b):
            return (
                pltpu.make_async_copy(rbufs[b], gx_hbm.at[i1m.at[j]],
                                      sems[b]),
                pltpu.make_async_copy(rbufs[b], gx_hbm.at[i2m.at[j]],
                                      sems[b]),
            )

        load_desc(0, 0).start()
        for j in range(_NCHD):
            b = j % 2
            load_desc(j, b).wait()
            if j >= 1:
                for c in scat_descs(j - 1, 1 - b):
                    c.wait()
            if j + 1 < _NCHD:
                load_desc(j + 1, 1 - b).start()
            for c in scat_descs(j, b):
                c.start()
        for c in scat_descs(_NCHD - 1, (_NCHD - 1) % 2):
            c.wait()

    return k(x_flat, d1m, d2m)


_CHC = 16
_NCHC = _TPW // _CHC


def _sc_combine(gy, dst1, dst2):
    @functools.partial(
        pl.kernel, mesh=_sc_mesh(),
        out_type=[
            jax.ShapeDtypeStruct((_T, _D), jnp.float32),
            jax.ShapeDtypeStruct((_T, _D), jnp.float32),
        ],
        scratch_types=[
            pltpu.VMEM((_TPW,), jnp.int32),
            pltpu.VMEM((_TPW,), jnp.int32),
            pltpu.VMEM((_CHC, _D), jnp.float32),
            pltpu.VMEM((_CHC, _D), jnp.float32),
            pltpu.VMEM((_CHC, _D), jnp.float32),
            pltpu.VMEM((_CHC, _D), jnp.float32),
            pltpu.SemaphoreType.DMA,
            pltpu.SemaphoreType.DMA,
            pltpu.SemaphoreType.DMA,
            pltpu.SemaphoreType.DMA,
        ],
    )
    def k(gy_hbm, d1_hbm, d2_hbm, g1_hbm, g2_hbm,
          i1_v, i2_v, r1a, r2a, r1b, r2b, semla, semlb, semwa, semwb):
        wid = lax.axis_index("s") * 2 + lax.axis_index("c")
        base = wid * _TPW
        pltpu.sync_copy(d1_hbm.at[pl.ds(base, _TPW)], i1_v)
        pltpu.sync_copy(d2_hbm.at[pl.ds(base, _TPW)], i2_v)
        r1 = (r1a, r1b)
        r2 = (r2a, r2b)
        seml = (semla, semlb)
        semw = (semwa, semwb)

        def load_descs(j, b):
            sl = pl.ds(j * _CHC, _CHC)
            return (
                pltpu.make_async_copy(gy_hbm.at[i1_v.at[sl]], r1[b],
                                      seml[b]),
                pltpu.make_async_copy(gy_hbm.at[i2_v.at[sl]], r2[b],
                                      seml[b]),
            )

        def write_descs(j, b):
            sl = pl.ds(base + j * _CHC, _CHC)
            return (
                pltpu.make_async_copy(r1[b], g1_hbm.at[sl], semw[b]),
                pltpu.make_async_copy(r2[b], g2_hbm.at[sl], semw[b]),
            )

        for c in load_descs(0, 0):
            c.start()
        for j in range(_NCHC):
            b = j % 2
            for c in load_descs(j, b):
                c.wait()
            if j >= 1:
                for c in write_descs(j - 1, 1 - b):
                    c.wait()
            if j + 1 < _NCHC:
                for c in load_descs(j + 1, 1 - b):
                    c.start()
            for c in write_descs(j, b):
                c.start()
        for c in write_descs(_NCHC - 1, (_NCHC - 1) % 2):
            c.wait()

    return k(gy, dst1, dst2)


def _shared_body(x_ref, w1_ref, b1_ref, w2_ref, b2_ref, o_ref):
    x = x_ref[...]
    h = lax.dot_general(x, w1_ref[...], (((1,), (1,)), ((), ())),
                        preferred_element_type=jnp.float32)
    h = _gelu(h + b1_ref[...])
    sh = lax.dot_general(h, w2_ref[...], (((1,), (1,)), ((), ())),
                         preferred_element_type=jnp.float32)
    o_ref[...] = sh + b2_ref[...]


def _fuse_body(sh_ref, g1_ref, g2_ref, s1_ref, s2_ref, o_ref):
    s1 = s1_ref[:, 0:1]
    s2 = s2_ref[:, 0:1]
    o_ref[...] = sh_ref[...] + s1 * g1_ref[...] + s2 * g2_ref[...]


def _fuse(sh, g1, g2, s1x, s2x):
    return pl.pallas_call(
        _fuse_body,
        grid=(_NTT,),
        in_specs=[
            pl.BlockSpec((_TT, _D), lambda i: (i, 0)),
            pl.BlockSpec((_TT, _D), lambda i: (i, 0)),
            pl.BlockSpec((_TT, _D), lambda i: (i, 0)),
            pl.BlockSpec((_TT, 16), lambda i: (i, 0)),
            pl.BlockSpec((_TT, 16), lambda i: (i, 0)),
        ],
        out_specs=pl.BlockSpec((_TT, _D), lambda i: (i, 0)),
        out_shape=jax.ShapeDtypeStruct((_T, _D), jnp.float32),
    )(sh, g1, g2, s1x, s2x)


def _shared(x_flat, sw1, sb1, sw2, sb2):
    return pl.pallas_call(
        _shared_body,
        grid=(_NTT,),
        in_specs=[
            pl.BlockSpec((_TT, _D), lambda i: (i, 0)),
            pl.BlockSpec((_H, _D), lambda i: (0, 0)),
            pl.BlockSpec((1, _H), lambda i: (0, 0)),
            pl.BlockSpec((_D, _H), lambda i: (0, 0)),
            pl.BlockSpec((1, _D), lambda i: (0, 0)),
        ],
        out_specs=pl.BlockSpec((_TT, _D), lambda i: (i, 0)),
        out_shape=jax.ShapeDtypeStruct((_T, _D), jnp.float32),
    )(x_flat, sw1, sb1, sw2, sb2)


def kernel(x, shared_fc1_w, shared_fc1_b, shared_fc2_w, shared_fc2_b,
           expert_fc1_w, expert_fc1_b, expert_fc2_w, expert_fc2_b, gate_w):
    x_flat = x.reshape(_T, _D)
    i1c, i2c, s1x, s2x, pp = _router(x_flat, gate_w)
    dst1, dst2, te, nv, freq, prob, loss = _plan(i1c, i2c, pp)
    dst1 = dst1.reshape(_T)
    dst2 = dst2.reshape(_T)

    gx = _sc_dispatch(x_flat, dst1.reshape(_NW, _NCHD, _CH),
                      dst2.reshape(_NW, _NCHD, _CH))

    sh = _shared(x_flat, shared_fc1_w, shared_fc1_b.reshape(1, _H),
                 shared_fc2_w, shared_fc2_b.reshape(1, _D))

    gy = _expert_mlp(
        te.reshape(_NT), nv.reshape(1), gx,
        expert_fc1_w, expert_fc1_b.reshape(_E, 1, _H),
        expert_fc2_w, expert_fc2_b.reshape(_E, 1, _D))

    g1, g2 = _sc_combine(gy, dst1, dst2)
    out = _fuse(sh, g1, g2, s1x, s2x)
    return (out.reshape(_B, _N, _D), loss.reshape(()), freq.reshape(_E),
            prob.reshape(_E))
